# baseline (speedup 1.0000x reference)
import functools

import jax
import jax.numpy as jnp
from jax import lax
from jax.experimental import pallas as pl
from jax.experimental.pallas import tpu as pltpu
from jax.experimental.pallas import tpu_sc as plsc

N = 10000
E = 320000
D = 128
WPAD = 144
PPAD = 16
NC = 2
NS = 16
NW = NC * NS
EW = E // NW
CB = 40
NCH = EW // CB
ROWB = 1000
RPT = N // NS
DB = 2000

_HI = jax.lax.Precision.HIGHEST


def _dot(a, b):
    return jax.lax.dot_general(a, b, (((1,), (0,)), ((), ())),
                               precision=_HI, preferred_element_type=jnp.float32)


def _dotT(a, w):
    return jax.lax.dot_general(a, w, (((1,), (1,)), ((), ())),
                               precision=_HI, preferred_element_type=jnp.float32)


def _dotTl(w, b):
    return jax.lax.dot_general(w, b, (((0,), (0,)), ((), ())),
                               precision=_HI, preferred_element_type=jnp.float32)


def _k1_body(x_ref, o_ref):
    x = x_ref[...]
    nrm = jnp.sqrt(jnp.sum(x * x, axis=1, keepdims=True))
    xn = x / jnp.maximum(nrm, 1e-12)
    ones = jnp.ones((x.shape[0], 1), jnp.float32)
    zeros = jnp.zeros((x.shape[0], WPAD - D - 1), jnp.float32)
    o_ref[...] = jnp.concatenate([xn, ones, zeros], axis=1)


_k1 = pl.pallas_call(
    _k1_body,
    grid=(N // ROWB,),
    in_specs=[pl.BlockSpec((ROWB, D), lambda i: (i, 0))],
    out_specs=pl.BlockSpec((ROWB, WPAD), lambda i: (i, 0)),
    out_shape=jax.ShapeDtypeStruct((N, WPAD), jnp.float32),
)


def _make_segsum(width):
    mesh = plsc.VectorSubcoreMesh(core_axis_name="c", subcore_axis_name="s")

    @functools.partial(
        pl.kernel,
        out_type=jax.ShapeDtypeStruct((NC, N, width), jnp.float32),
        mesh=mesh,
        compiler_params=pltpu.CompilerParams(use_tc_tiling_on_sc=False),
        scratch_types=[
            pltpu.VMEM((NCH, CB), jnp.int32),
            pltpu.VMEM((NCH, CB), jnp.int32),
            pltpu.VMEM((CB, width), jnp.float32),
            pltpu.VMEM((CB, width), jnp.float32),
            pltpu.VMEM_SHARED((N, width), jnp.float32),
            pltpu.SemaphoreType.DMA,
            pltpu.SemaphoreType.DMA,
            pltpu.SemaphoreType.DMA,
            pltpu.SemaphoreType.DMA,
        ],
    )
    def k(tab_hbm, src_hbm, dst_hbm, zero_hbm, out_hbm,
          sidx, didx, rows0, rows1, acc, g0, g1, s0, s1):
        cid = lax.axis_index("c")
        sid = lax.axis_index("s")
        wid = sid * NC + cid
        r0 = sid * RPT
        pltpu.sync_copy(zero_hbm.at[pl.ds(r0, RPT)], acc.at[pl.ds(r0, RPT)])
        pltpu.sync_copy(src_hbm.at[wid], sidx)
        pltpu.sync_copy(dst_hbm.at[wid], didx)
        plsc.subcore_barrier()

        def gather(c, buf, sem):
            pltpu.make_async_copy(tab_hbm.at[sidx.at[c]], buf, sem).start()

        def gwait(buf, sem):
            pltpu.make_async_copy(tab_hbm.at[sidx.at[0]], buf, sem).wait()

        def scat(c, buf, sem):
            pltpu.make_async_copy(buf, acc.at[didx.at[c]], sem).start(add=True)

        def swait(c, buf, sem):
            pltpu.make_async_copy(buf, acc.at[didx.at[c]], sem).wait()

        gather(0, rows0, g0)
        gather(1, rows1, g1)

        def body(j, carry):
            c = 2 * j
            gwait(rows0, g0)
            scat(c, rows0, s0)
            gwait(rows1, g1)
            scat(c + 1, rows1, s1)
            swait(c, rows0, s0)
            gather(jnp.minimum(c + 2, NCH - 1), rows0, g0)
            swait(c + 1, rows1, s1)
            gather(jnp.minimum(c + 3, NCH - 1), rows1, g1)
            return carry

        lax.fori_loop(0, NCH // 2, body, 0)
        gwait(rows1, g1)
        gwait(rows0, g0)
        plsc.subcore_barrier()
        pltpu.sync_copy(acc.at[pl.ds(r0, RPT)], out_hbm.at[cid, pl.ds(r0, RPT)])

    return k


_seg144 = _make_segsum(WPAD)
_seg16 = _make_segsum(PPAD)


def _k3_body(P_ref, xn_ref, W1l_ref, b1l_ref, W1r_ref, W2l_ref, b2l_ref,
             W2r_ref, wst_ref, p_ref, meta_ref):
    P = P_ref[...]
    Msum = P[0] + P[1]
    degc = jnp.maximum(Msum[:, D:D + 1], 1.0)
    agg = Msum[:, :D] / degc
    xn = xn_ref[:, :D]
    h = jnp.maximum(_dotT(agg, W1l_ref[...]) + b1l_ref[...]
                    + _dotT(xn, W1r_ref[...]), 0.0)
    wst = wst_ref[...]
    B1 = _dotTl(W2l_ref[...], wst)
    B2 = _dotTl(W2r_ref[...], wst)
    c2 = _dot(b2l_ref[...], wst)
    p2 = _dot(h, B1)
    t2 = _dot(h, B2) + c2
    m = p2.shape[0]
    p_ref[...] = jnp.concatenate(
        [p2, jnp.zeros((m, PPAD - 2), jnp.float32)], axis=1)
    meta_ref[...] = jnp.concatenate(
        [t2, degc, jnp.zeros((m, 5), jnp.float32)], axis=1)


_k3 = pl.pallas_call(
    _k3_body,
    grid=(N // ROWB,),
    in_specs=[
        pl.BlockSpec((NC, ROWB, WPAD), lambda i: (0, i, 0)),
        pl.BlockSpec((ROWB, WPAD), lambda i: (i, 0)),
        pl.BlockSpec((D, D), lambda i: (0, 0)),
        pl.BlockSpec((1, D), lambda i: (0, 0)),
        pl.BlockSpec((D, D), lambda i: (0, 0)),
        pl.BlockSpec((D, D), lambda i: (0, 0)),
        pl.BlockSpec((1, D), lambda i: (0, 0)),
        pl.BlockSpec((D, D), lambda i: (0, 0)),
        pl.BlockSpec((D, 2), lambda i: (0, 0)),
    ],
    out_specs=[
        pl.BlockSpec((ROWB, PPAD), lambda i: (i, 0)),
        pl.BlockSpec((ROWB, 8), lambda i: (i, 0)),
    ],
    out_shape=[
        jax.ShapeDtypeStruct((N, PPAD), jnp.float32),
        jax.ShapeDtypeStruct((N, 8), jnp.float32),
    ],
)


def _k5_body(S_ref, meta_ref, blin_ref, ab_ref):
    S = S_ref[...]
    s = S[0] + S[1]
    meta = meta_ref[...]
    degc = meta[:, 2:3]
    a = s[:, 0:1] / degc + meta[:, 0:1] + blin_ref[0, 0]
    b = s[:, 1:2] / degc + meta[:, 1:2]
    ab_ref[...] = jnp.concatenate([a, b], axis=1)


_k5 = pl.pallas_call(
    _k5_body,
    grid=(N // ROWB,),
    in_specs=[
        pl.BlockSpec((NC, ROWB, PPAD), lambda i: (0, i, 0)),
        pl.BlockSpec((ROWB, 8), lambda i: (i, 0)),
        pl.BlockSpec((1, 1), lambda i: (0, 0)),
    ],
    out_specs=pl.BlockSpec((ROWB, 2), lambda i: (i, 0)),
    out_shape=jax.ShapeDtypeStruct((N, 2), jnp.float32),
)


def _make_decode():
    mesh = plsc.VectorSubcoreMesh(core_axis_name="c", subcore_axis_name="s")

    @functools.partial(
        pl.kernel,
        out_type=jax.ShapeDtypeStruct((E,), jnp.float32),
        mesh=mesh,
        compiler_params=pltpu.CompilerParams(use_tc_tiling_on_sc=False,
                                             needs_layout_passes=False),
        scratch_types=[
            pltpu.VMEM((N, 2), jnp.float32),
            pltpu.VMEM((DB,), jnp.int32),
            pltpu.VMEM((DB,), jnp.int32),
            pltpu.VMEM((DB,), jnp.float32),
        ],
    )
    def k(ab_hbm, src_hbm, dst_hbm, out_hbm, abv, sv, dv, ov):
        cid = lax.axis_index("c")
        sid = lax.axis_index("s")
        wid = sid * NC + cid
        base = wid * EW
        pltpu.sync_copy(ab_hbm, abv)
        col0 = jnp.zeros((16,), jnp.int32)
        col1 = jnp.ones((16,), jnp.int32)
        for j in range(EW // DB):
            off = base + j * DB
            pltpu.sync_copy(src_hbm.at[pl.ds(off, DB)], sv)
            pltpu.sync_copy(dst_hbm.at[pl.ds(off, DB)], dv)

            def body(i, carry):
                s16 = sv[pl.ds(i * 16, 16)]
                d16 = dv[pl.ds(i * 16, 16)]
                av = plsc.load_gather(abv, [s16, col0])
                bv = plsc.load_gather(abv, [d16, col1])
                t = av + bv
                ov[pl.ds(i * 16, 16)] = 1.0 / (1.0 + jnp.exp(-t))
                return carry

            lax.fori_loop(0, DB // 16, body, 0)
            pltpu.sync_copy(ov, out_hbm.at[pl.ds(off, DB)])

    return k


_decode = _make_decode()


def kernel(x, edge_index, W1l, b1l, W1r, W2l, b2l, W2r, Wlin, blin):
    src = edge_index[0]
    dst = edge_index[1]
    src2 = src.reshape(NW, NCH, CB)
    dst2 = dst.reshape(NW, NCH, CB)
    xn_pad = _k1(x)
    zeros_w = jnp.zeros((N, WPAD), jnp.float32)
    zeros_p = jnp.zeros((N, PPAD), jnp.float32)
    P = _seg144(xn_pad, src2, dst2, zeros_w)
    wst = jnp.transpose(Wlin.reshape(2, D))
    p_pad, meta = _k3(P, xn_pad, W1l, b1l.reshape(1, D), W1r, W2l,
                      b2l.reshape(1, D), W2r, wst)
    S = _seg16(p_pad, src2, dst2, zeros_p)
    ab = _k5(S, meta, blin.reshape(1, 1))
    out = _decode(ab, src, dst)
    return out.reshape(E, 1)


# speedup vs baseline: 1.2493x; 1.2493x over previous
import functools

import jax
import jax.numpy as jnp
from jax import lax
from jax.experimental import pallas as pl
from jax.experimental.pallas import tpu as pltpu
from jax.experimental.pallas import tpu_sc as plsc

N = 10000
E = 320000
D = 128
WPAD = 144
PPAD = 16
NC = 2
NS = 16
NW = NC * NS
EW = E // NW
CBW = 40
CBN = 100
ROWB = 1000
RPT = N // NS
DB = 2000

_HI = jax.lax.Precision.HIGHEST


def _dot(a, b):
    return jax.lax.dot_general(a, b, (((1,), (0,)), ((), ())),
                               precision=_HI, preferred_element_type=jnp.float32)


def _dotT(a, w):
    return jax.lax.dot_general(a, w, (((1,), (1,)), ((), ())),
                               precision=_HI, preferred_element_type=jnp.float32)


def _dotTl(w, b):
    return jax.lax.dot_general(w, b, (((0,), (0,)), ((), ())),
                               precision=_HI, preferred_element_type=jnp.float32)


def _k1_body(x_ref, o_ref):
    x = x_ref[...]
    nrm = jnp.sqrt(jnp.sum(x * x, axis=1, keepdims=True))
    xn = x / jnp.maximum(nrm, 1e-12)
    ones = jnp.ones((x.shape[0], 1), jnp.float32)
    zeros = jnp.zeros((x.shape[0], WPAD - D - 1), jnp.float32)
    o_ref[...] = jnp.concatenate([xn, ones, zeros], axis=1)


_k1 = pl.pallas_call(
    _k1_body,
    grid=(N // ROWB,),
    in_specs=[pl.BlockSpec((ROWB, D), lambda i: (i, 0))],
    out_specs=pl.BlockSpec((ROWB, WPAD), lambda i: (i, 0)),
    out_shape=jax.ShapeDtypeStruct((N, WPAD), jnp.float32),
)


def _make_segsum(width, cb):
    nch = EW // cb
    assert nch % 2 == 0
    mesh = plsc.VectorSubcoreMesh(core_axis_name="c", subcore_axis_name="s")

    @functools.partial(
        pl.kernel,
        out_type=jax.ShapeDtypeStruct((NC, N, width), jnp.float32),
        mesh=mesh,
        compiler_params=pltpu.CompilerParams(use_tc_tiling_on_sc=False),
        scratch_types=[
            pltpu.VMEM((nch, cb), jnp.int32),
            pltpu.VMEM((nch, cb), jnp.int32),
            pltpu.VMEM((cb, width), jnp.float32),
            pltpu.VMEM((cb, width), jnp.float32),
            pltpu.VMEM_SHARED((N, width), jnp.float32),
            pltpu.SemaphoreType.DMA,
            pltpu.SemaphoreType.DMA,
        ],
    )
    def k(tab_hbm, src_hbm, dst_hbm, zero_hbm, out_hbm,
          sidx, didx, rows0, rows1, acc, g0, g1):
        cid = lax.axis_index("c")
        sid = lax.axis_index("s")
        wid = sid * NC + cid
        r0 = sid * RPT
        pltpu.sync_copy(zero_hbm.at[pl.ds(r0, RPT)], acc.at[pl.ds(r0, RPT)])
        pltpu.sync_copy(src_hbm.at[wid], sidx)
        pltpu.sync_copy(dst_hbm.at[wid], didx)
        plsc.subcore_barrier()

        def gather(c, buf, sem):
            pltpu.make_async_copy(tab_hbm.at[sidx.at[c]], buf, sem).start()

        def gwait(buf, sem):
            pltpu.make_async_copy(tab_hbm.at[sidx.at[0]], buf, sem).wait()

        def scat(c, buf):
            pltpu.sync_copy(buf, acc.at[didx.at[c]], add=True)

        gather(0, rows0, g0)
        gather(1, rows1, g1)

        def body(j, carry):
            c = 2 * j
            gwait(rows0, g0)
            scat(c, rows0)
            gather(jnp.minimum(c + 2, nch - 1), rows0, g0)
            gwait(rows1, g1)
            scat(c + 1, rows1)
            gather(jnp.minimum(c + 3, nch - 1), rows1, g1)
            return carry

        lax.fori_loop(0, nch // 2, body, 0)
        gwait(rows1, g1)
        gwait(rows0, g0)
        plsc.subcore_barrier()
        pltpu.sync_copy(acc.at[pl.ds(r0, RPT)], out_hbm.at[cid, pl.ds(r0, RPT)])

    return k


_seg144 = _make_segsum(WPAD, CBW)
_seg16 = _make_segsum(PPAD, CBN)


def _k3_body(P_ref, xn_ref, W1l_ref, b1l_ref, W1r_ref, W2l_ref, b2l_ref,
             W2r_ref, wst_ref, blin_ref, p_ref, meta_ref):
    P = P_ref[...]
    Msum = P[0] + P[1]
    degc = jnp.maximum(Msum[:, D:D + 1], 1.0)
    agg = Msum[:, :D] / degc
    xn = xn_ref[:, :D]
    h = jnp.maximum(_dotT(agg, W1l_ref[...]) + b1l_ref[...]
                    + _dotT(xn, W1r_ref[...]), 0.0)
    wst = wst_ref[...]
    B1 = _dotTl(W2l_ref[...], wst)
    B2 = _dotTl(W2r_ref[...], wst)
    c2 = _dot(b2l_ref[...], wst)
    p2 = _dot(h, B1)
    t2 = _dot(h, B2) + c2
    m = p2.shape[0]
    p_ref[...] = jnp.concatenate(
        [p2, jnp.zeros((m, PPAD - 2), jnp.float32)], axis=1)
    meta_ref[...] = jnp.concatenate(
        [t2[:, 0:1] + blin_ref[0, 0], t2[:, 1:2], degc,
         jnp.zeros((m, 5), jnp.float32)], axis=1)


_k3 = pl.pallas_call(
    _k3_body,
    grid=(N // ROWB,),
    in_specs=[
        pl.BlockSpec((NC, ROWB, WPAD), lambda i: (0, i, 0)),
        pl.BlockSpec((ROWB, WPAD), lambda i: (i, 0)),
        pl.BlockSpec((D, D), lambda i: (0, 0)),
        pl.BlockSpec((1, D), lambda i: (0, 0)),
        pl.BlockSpec((D, D), lambda i: (0, 0)),
        pl.BlockSpec((D, D), lambda i: (0, 0)),
        pl.BlockSpec((1, D), lambda i: (0, 0)),
        pl.BlockSpec((D, D), lambda i: (0, 0)),
        pl.BlockSpec((D, 2), lambda i: (0, 0)),
        pl.BlockSpec((1, 1), lambda i: (0, 0)),
    ],
    out_specs=[
        pl.BlockSpec((ROWB, PPAD), lambda i: (i, 0)),
        pl.BlockSpec((ROWB, 8), lambda i: (i, 0)),
    ],
    out_shape=[
        jax.ShapeDtypeStruct((N, PPAD), jnp.float32),
        jax.ShapeDtypeStruct((N, 8), jnp.float32),
    ],
)


def _make_decode():
    mesh = plsc.VectorSubcoreMesh(core_axis_name="c", subcore_axis_name="s")

    @functools.partial(
        pl.kernel,
        out_type=jax.ShapeDtypeStruct((E,), jnp.float32),
        mesh=mesh,
        compiler_params=pltpu.CompilerParams(use_tc_tiling_on_sc=False,
                                             needs_layout_passes=False),
        scratch_types=[
            pltpu.VMEM((RPT, PPAD), jnp.float32),
            pltpu.VMEM((RPT, PPAD), jnp.float32),
            pltpu.VMEM((RPT, 8), jnp.float32),
            pltpu.VMEM((RPT, 2), jnp.float32),
            pltpu.VMEM_SHARED((N, 2), jnp.float32),
            pltpu.VMEM((N, 2), jnp.float32),
            pltpu.VMEM((DB,), jnp.int32),
            pltpu.VMEM((DB,), jnp.int32),
            pltpu.VMEM((DB,), jnp.float32),
        ],
    )
    def k(S_hbm, meta_hbm, src_hbm, dst_hbm, out_hbm,
          s0v, s1v, metav, absl, absh, abv, sv, dv, ov):
        cid = lax.axis_index("c")
        sid = lax.axis_index("s")
        wid = sid * NC + cid
        base = wid * EW
        r0 = sid * RPT
        col0 = jnp.zeros((16,), jnp.int32)
        col1 = jnp.ones((16,), jnp.int32)
        col2 = col1 + col1
        pltpu.sync_copy(S_hbm.at[0, pl.ds(r0, RPT)], s0v)
        pltpu.sync_copy(S_hbm.at[1, pl.ds(r0, RPT)], s1v)
        pltpu.sync_copy(meta_hbm.at[pl.ds(r0, RPT)], metav)
        riota = lax.iota(jnp.int32, 16)

        def cbody(i, carry):
            r16 = jnp.minimum(i * 16, RPT - 16) + riota
            sa = (plsc.load_gather(s0v, [r16, col0])
                  + plsc.load_gather(s1v, [r16, col0]))
            sb = (plsc.load_gather(s0v, [r16, col1])
                  + plsc.load_gather(s1v, [r16, col1]))
            ta = plsc.load_gather(metav, [r16, col0])
            tb = plsc.load_gather(metav, [r16, col1])
            dg = plsc.load_gather(metav, [r16, col2])
            plsc.store_scatter(absl, [r16, col0], sa / dg + ta)
            plsc.store_scatter(absl, [r16, col1], sb / dg + tb)
            return carry

        lax.fori_loop(0, (RPT + 15) // 16, cbody, 0)
        pltpu.sync_copy(absl, absh.at[pl.ds(r0, RPT)])
        plsc.subcore_barrier()
        pltpu.sync_copy(absh, abv)
        for j in range(EW // DB):
            off = base + j * DB
            pltpu.sync_copy(src_hbm.at[pl.ds(off, DB)], sv)
            pltpu.sync_copy(dst_hbm.at[pl.ds(off, DB)], dv)

            def body(i, carry):
                s16 = sv[pl.ds(i * 16, 16)]
                d16 = dv[pl.ds(i * 16, 16)]
                av = plsc.load_gather(abv, [s16, col0])
                bv = plsc.load_gather(abv, [d16, col1])
                t = av + bv
                ov[pl.ds(i * 16, 16)] = 1.0 / (1.0 + jnp.exp(-t))
                return carry

            lax.fori_loop(0, DB // 16, body, 0)
            pltpu.sync_copy(ov, out_hbm.at[pl.ds(off, DB)])

    return k


_decode = _make_decode()


def kernel(x, edge_index, W1l, b1l, W1r, W2l, b2l, W2r, Wlin, blin):
    src = edge_index[0]
    dst = edge_index[1]
    xn_pad = _k1(x)
    zeros_w = jnp.zeros((N, WPAD), jnp.float32)
    zeros_p = jnp.zeros((N, PPAD), jnp.float32)
    P = _seg144(xn_pad, src.reshape(NW, EW // CBW, CBW),
                dst.reshape(NW, EW // CBW, CBW), zeros_w)
    wst = jnp.transpose(Wlin.reshape(2, D))
    p_pad, meta = _k3(P, xn_pad, W1l, b1l.reshape(1, D), W1r, W2l,
                      b2l.reshape(1, D), W2r, wst, blin.reshape(1, 1))
    S = _seg16(p_pad, src.reshape(NW, EW // CBN, CBN),
               dst.reshape(NW, EW // CBN, CBN), zeros_p)
    out = _decode(S, meta, src, dst)
    return out.reshape(E, 1)
